# baseline (speedup 1.0000x reference)
import functools

import jax
import jax.numpy as jnp
from jax import lax
from jax.experimental import pallas as pl
from jax.experimental.pallas import tpu as pltpu
from jax.experimental.pallas import tpu_sc as plsc

BATCH = 16384
EMBED_DIM = 64

_INFO = plsc.get_sparse_core_info()
_NC = _INFO.num_cores
_NS = _INFO.num_subcores
_L = _INFO.num_lanes
_NW = _NC * _NS
_BPW = BATCH // _NW
_CHUNK = 16
_NCHUNK = _BPW // _CHUNK


@functools.partial(
    pl.kernel,
    mesh=plsc.VectorSubcoreMesh(core_axis_name="c", subcore_axis_name="s"),
    compiler_params=pltpu.CompilerParams(needs_layout_passes=False),
    out_type=jax.ShapeDtypeStruct((BATCH,), jnp.float32),
    scratch_types=[
        pltpu.VMEM((_BPW,), jnp.int32),
        pltpu.VMEM((_BPW,), jnp.int32),
        pltpu.VMEM((4, _CHUNK, EMBED_DIM), jnp.float32),
        pltpu.VMEM((4, _CHUNK, EMBED_DIM), jnp.float32),
        pltpu.VMEM((_BPW,), jnp.float32),
        pltpu.SemaphoreType.DMA,
        pltpu.SemaphoreType.DMA,
        pltpu.SemaphoreType.DMA,
        pltpu.SemaphoreType.DMA,
    ],
)
def _sc_dot_kernel(uids_hbm, mids_hbm, utab_hbm, mtab_hbm, out_hbm,
                   uidx_v, midx_v, urows_v, mrows_v, out_v,
                   sem0, sem1, sem2, sem3):
    wid = lax.axis_index("s") * _NC + lax.axis_index("c")
    base = wid * _BPW
    sems = (sem0, sem1, sem2, sem3)

    pltpu.sync_copy(uids_hbm.at[pl.ds(base, _BPW)], uidx_v)
    pltpu.sync_copy(mids_hbm.at[pl.ds(base, _BPW)], midx_v)

    def fire(c, buf):
        off = pl.multiple_of(c * _CHUNK, _L)
        u16 = uidx_v[pl.ds(off, _L)]
        m16 = midx_v[pl.ds(off, _L)]
        for j in range(_L):
            pltpu.async_copy(
                utab_hbm.at[u16[j]], urows_v.at[buf, j], sems[buf])
            pltpu.async_copy(
                mtab_hbm.at[m16[j]], mrows_v.at[buf, j], sems[buf])

    def drain(buf):
        for j in range(_L):
            pltpu.make_async_copy(
                utab_hbm.at[0], urows_v.at[buf, j], sems[buf]).wait()
            pltpu.make_async_copy(
                mtab_hbm.at[0], mrows_v.at[buf, j], sems[buf]).wait()

    def compute(c, buf):
        off = pl.multiple_of(c * _CHUNK, _L)
        slots = lax.iota(jnp.int32, _L)
        acc = jnp.zeros((_L,), jnp.float32)
        for k in range(EMBED_DIM):
            cols = jnp.full((_L,), k, jnp.int32)
            u = plsc.load_gather(urows_v.at[buf], [slots, cols])
            m = plsc.load_gather(mrows_v.at[buf], [slots, cols])
            acc = acc + u * m
        out_v[pl.ds(off, _L)] = acc

    for b in range(4):
        fire(b, b)

    @pl.loop(0, _NCHUNK - 4, step=4)
    def _ring(c):
        for b in range(4):
            drain(b)
            compute(c + b, b)
            fire(c + 4 + b, b)

    for b in range(4):
        drain(b)
        compute(_NCHUNK - 4 + b, b)

    pltpu.sync_copy(out_v, out_hbm.at[pl.ds(base, _BPW)])


def kernel(user_ids, movie_ids, user_table, movie_table):
    uids = user_ids.astype(jnp.int32)
    mids = movie_ids.astype(jnp.int32)
    return _sc_dot_kernel(uids, mids, user_table, movie_table)


# speedup vs baseline: 1.0104x; 1.0104x over previous
import functools

import jax
import jax.numpy as jnp
from jax import lax
from jax.experimental import pallas as pl
from jax.experimental.pallas import tpu as pltpu
from jax.experimental.pallas import tpu_sc as plsc

BATCH = 16384
EMBED_DIM = 64

_INFO = plsc.get_sparse_core_info()
_NC = _INFO.num_cores
_NS = _INFO.num_subcores
_L = _INFO.num_lanes
_NW = _NC * _NS
_BPW = BATCH // _NW
_CHUNK = 16
_NCHUNK = _BPW // _CHUNK


@functools.partial(
    pl.kernel,
    mesh=plsc.VectorSubcoreMesh(core_axis_name="c", subcore_axis_name="s"),
    compiler_params=pltpu.CompilerParams(needs_layout_passes=False),
    out_type=jax.ShapeDtypeStruct((BATCH,), jnp.float32),
    scratch_types=[
        pltpu.VMEM((_BPW,), jnp.int32),
        pltpu.VMEM((_BPW,), jnp.int32),
        pltpu.VMEM((2, _CHUNK, EMBED_DIM), jnp.float32),
        pltpu.VMEM((2, _CHUNK, EMBED_DIM), jnp.float32),
        pltpu.VMEM((_BPW,), jnp.float32),
        pltpu.SemaphoreType.DMA,
        pltpu.SemaphoreType.DMA,
    ],
)
def _sc_dot_kernel(uids_hbm, mids_hbm, utab_hbm, mtab_hbm, out_hbm,
                   uidx_v, midx_v, urows_v, mrows_v, out_v, sem0, sem1):
    wid = lax.axis_index("s") * _NC + lax.axis_index("c")
    base = wid * _BPW
    sems = (sem0, sem1)

    pltpu.sync_copy(uids_hbm.at[pl.ds(base, _BPW)], uidx_v)
    pltpu.sync_copy(mids_hbm.at[pl.ds(base, _BPW)], midx_v)

    def fire(c, buf):
        off = pl.multiple_of(c * _CHUNK, _L)
        u16 = uidx_v[pl.ds(off, _L)]
        m16 = midx_v[pl.ds(off, _L)]
        for j in range(_L):
            pltpu.async_copy(
                utab_hbm.at[u16[j]], urows_v.at[buf, j], sems[buf])
            pltpu.async_copy(
                mtab_hbm.at[m16[j]], mrows_v.at[buf, j], sems[buf])

    def drain(buf):
        for j in range(_L):
            pltpu.make_async_copy(
                utab_hbm.at[0], urows_v.at[buf, j], sems[buf]).wait()
            pltpu.make_async_copy(
                mtab_hbm.at[0], mrows_v.at[buf, j], sems[buf]).wait()

    def compute(c, buf):
        off = pl.multiple_of(c * _CHUNK, _L)
        slots = lax.iota(jnp.int32, _L)
        acc = jnp.zeros((_L,), jnp.float32)
        for k in range(EMBED_DIM):
            cols = jnp.full((_L,), k, jnp.int32)
            u = plsc.load_gather(urows_v.at[buf], [slots, cols])
            m = plsc.load_gather(mrows_v.at[buf], [slots, cols])
            acc = acc + u * m
        out_v[pl.ds(off, _L)] = acc

    fire(0, 0)
    fire(1, 1)

    @pl.loop(0, _NCHUNK - 2, step=2)
    def _ring(c):
        drain(0)
        compute(c, 0)
        fire(c + 2, 0)
        drain(1)
        compute(c + 1, 1)
        fire(c + 3, 1)

    drain(0)
    compute(_NCHUNK - 2, 0)
    drain(1)
    compute(_NCHUNK - 1, 1)

    pltpu.sync_copy(out_v, out_hbm.at[pl.ds(base, _BPW)])


def kernel(user_ids, movie_ids, user_table, movie_table):
    uids = user_ids.astype(jnp.int32)
    mids = movie_ids.astype(jnp.int32)
    return _sc_dot_kernel(uids, mids, user_table, movie_table)
